# baseline (speedup 1.0000x reference)
import functools

import jax
import jax.numpy as jnp
from jax import lax
from jax.experimental import pallas as pl
from jax.experimental.pallas import tpu as pltpu
from jax.experimental.pallas import tpu_sc as plsc

_D = 32
_NC, _NS = 2, 16
_NW = _NC * _NS
_IPC = 16
_NBUF = 2


@functools.partial(jax.jit, static_argnums=(2, 3))
def _embed_gather(idx, table, n_i, n_j):
    pj = -(-n_j // 8) * 8
    ch = _IPC * pj
    ipw = n_i // _NW
    bpw = ipw * pj
    nch = ipw // _IPC
    mesh = plsc.VectorSubcoreMesh(core_axis_name="c", subcore_axis_name="s")

    @functools.partial(
        pl.kernel,
        out_type=jax.ShapeDtypeStruct((n_i, n_j, _D), jnp.float32),
        mesh=mesh,
        scratch_types=[
            pltpu.VMEM((bpw,), jnp.int32),
            pltpu.VMEM((_NBUF, ch, _D), jnp.float32),
        ] + [pltpu.SemaphoreType.DMA] * (2 * _NBUF),
        compiler_params=pltpu.CompilerParams(use_tc_tiling_on_sc=False),
    )
    def k(idx_hbm, table_hbm, out_hbm, idx_v, rows_v, *sems):
        gsems, wsems = sems[:_NBUF], sems[_NBUF:]
        wid = lax.axis_index("s") * _NC + lax.axis_index("c")
        i_base = wid * ipw
        pltpu.sync_copy(idx_hbm.at[pl.ds(wid * bpw, bpw)], idx_v)

        def start_gather(buf, c):
            pltpu.async_copy(table_hbm.at[idx_v.at[pl.ds(c * ch, ch)]],
                             rows_v.at[buf], gsems[buf])

        def wait_gather(buf):
            pltpu.make_async_copy(table_hbm.at[idx_v.at[pl.ds(0, ch)]],
                                  rows_v.at[buf], gsems[buf]).wait()

        def fire_writes(buf, c):
            i0 = i_base + c * _IPC
            for kk in range(_IPC):
                pltpu.async_copy(rows_v.at[buf].at[pl.ds(kk * pj, n_j)],
                                 out_hbm.at[i0 + kk], wsems[buf])

        def drain_writes(buf):
            for _ in range(_IPC):
                pltpu.make_async_copy(rows_v.at[buf].at[pl.ds(0, n_j)],
                                      out_hbm.at[0], wsems[buf]).wait()

        for buf in range(_NBUF):
            start_gather(buf, buf)

        @pl.loop(0, nch, step=_NBUF)
        def _(t):
            for buf in range(_NBUF):
                c = t + buf
                wait_gather(buf)
                fire_writes(buf, c)

                @pl.when(c + _NBUF < nch)
                def _():
                    drain_writes(buf)
                    start_gather(buf, c + _NBUF)

        for buf in range(_NBUF):
            drain_writes(buf)

    return k(idx, table)


def kernel(x, table):
    n_i, n_j = x.shape
    pj = -(-n_j // 8) * 8
    xp = jnp.pad(x, ((0, 0), (0, pj - n_j))) if pj != n_j else x
    return _embed_gather(xp.reshape(-1), table, n_i, n_j)


# speedup vs baseline: 1.2725x; 1.2725x over previous
import functools

import jax
import jax.numpy as jnp
from jax import lax
from jax.experimental import pallas as pl
from jax.experimental.pallas import tpu as pltpu
from jax.experimental.pallas import tpu_sc as plsc

_D = 32
_NC, _NS = 2, 16
_NW = _NC * _NS
_CH = 800
_NBUF = 4


@jax.jit
def _embed_gather(idx, table):
    b = idx.shape[0]
    bpw = b // _NW
    nch = bpw // _CH
    mesh = plsc.VectorSubcoreMesh(core_axis_name="c", subcore_axis_name="s")

    @functools.partial(
        pl.kernel,
        out_type=jax.ShapeDtypeStruct((b, _D), jnp.float32),
        mesh=mesh,
        scratch_types=[
            pltpu.VMEM((bpw,), jnp.int32),
            pltpu.VMEM((_NBUF, _CH, _D), jnp.float32),
        ] + [pltpu.SemaphoreType.DMA] * _NBUF,
        compiler_params=pltpu.CompilerParams(use_tc_tiling_on_sc=False),
    )
    def k(idx_hbm, table_hbm, out_hbm, idx_v, rows_v, *sems):
        wid = lax.axis_index("s") * _NC + lax.axis_index("c")
        base = wid * bpw
        pltpu.sync_copy(idx_hbm.at[pl.ds(base, bpw)], idx_v)

        def start_gather(buf, c):
            pltpu.async_copy(table_hbm.at[idx_v.at[pl.ds(c * _CH, _CH)]],
                             rows_v.at[buf], sems[buf])

        def wait_gather(buf):
            pltpu.make_async_copy(table_hbm.at[idx_v.at[pl.ds(0, _CH)]],
                                  rows_v.at[buf], sems[buf]).wait()

        for buf in range(_NBUF):
            start_gather(buf, buf)

        @pl.loop(0, nch, step=_NBUF)
        def _(t):
            for buf in range(_NBUF):
                c = t + buf
                wait_gather(buf)
                pltpu.sync_copy(rows_v.at[buf],
                                out_hbm.at[pl.ds(base + c * _CH, _CH)])

                @pl.when(c + _NBUF < nch)
                def _():
                    start_gather(buf, c + _NBUF)

    return k(idx, table)


def kernel(x, table):
    shp = x.shape
    out = _embed_gather(x.reshape(-1), table)
    return out.reshape(*shp, table.shape[1])


# speedup vs baseline: 2.3507x; 1.8473x over previous
import functools

import jax
import jax.numpy as jnp
from jax import lax
from jax.experimental import pallas as pl
from jax.experimental.pallas import tpu as pltpu
from jax.experimental.pallas import tpu_sc as plsc

_D = 32
_NC, _NS = 2, 16
_NW = _NC * _NS
_IPC = 16
_NBUF = 2
_L = 16


@functools.partial(jax.jit, static_argnums=(2, 3))
def _embed_gather(idx, table, n_i, n_j):
    ch = _IPC * n_j
    row_w = n_j * _D
    ipw = n_i // _NW
    bpw = ipw * n_j
    nch = ipw // _IPC
    qn = (n_j * _D) // _L
    mesh = plsc.VectorSubcoreMesh(core_axis_name="c", subcore_axis_name="s")

    @functools.partial(
        pl.kernel,
        out_type=jax.ShapeDtypeStruct((n_i, row_w), jnp.float32),
        mesh=mesh,
        scratch_types=[
            pltpu.VMEM((bpw,), jnp.int32),
            pltpu.VMEM((_NBUF, ch, _D), jnp.float32),
            pltpu.VMEM((_NBUF, _IPC, row_w), jnp.float32),
        ] + [pltpu.SemaphoreType.DMA] * _NBUF,
        compiler_params=pltpu.CompilerParams(use_tc_tiling_on_sc=False),
    )
    def k(idx_hbm, table_hbm, out_hbm, idx_v, gbuf, wbuf, *sems):
        wid = lax.axis_index("s") * _NC + lax.axis_index("c")
        base = wid * bpw
        i_base = wid * ipw
        pltpu.sync_copy(idx_hbm.at[pl.ds(base, bpw)], idx_v)

        def start_gather(buf, c):
            pltpu.async_copy(table_hbm.at[idx_v.at[pl.ds(c * ch, ch)]],
                             gbuf.at[buf], sems[buf])

        def wait_gather(buf):
            pltpu.make_async_copy(table_hbm.at[idx_v.at[pl.ds(0, ch)]],
                                  gbuf.at[buf], sems[buf]).wait()

        def shuffle(buf):
            @pl.loop(0, qn)
            def _(q):
                w0 = q * _L
                r_in_i = lax.shift_right_logical(w0, 5)
                c0 = lax.bitwise_and(w0, 31)
                for i in range(_IPC):
                    v = gbuf[buf, i * n_j + r_in_i, pl.ds(c0, _L)]
                    wbuf[buf, i, pl.ds(w0, _L)] = v

        for buf in range(_NBUF):
            start_gather(buf, buf)

        @pl.loop(0, nch, step=_NBUF)
        def _(t):
            for buf in range(_NBUF):
                c = t + buf
                wait_gather(buf)
                shuffle(buf)
                pltpu.sync_copy(wbuf.at[buf],
                                out_hbm.at[pl.ds(i_base + c * _IPC, _IPC)])

                @pl.when(c + _NBUF < nch)
                def _():
                    start_gather(buf, c + _NBUF)

    return k(idx, table)


def kernel(x, table):
    n_i, n_j = x.shape
    out = _embed_gather(x.reshape(-1), table, n_i, n_j)
    return out.reshape(n_i, n_j, table.shape[1])


# speedup vs baseline: 2.7596x; 1.1739x over previous
import functools

import jax
import jax.numpy as jnp
from jax import lax
from jax.experimental import pallas as pl
from jax.experimental.pallas import tpu as pltpu
from jax.experimental.pallas import tpu_sc as plsc

_D = 32
_NC, _NS = 2, 16
_NW = _NC * _NS
_IPC = 16
_NBUF = 2
_L = 16


@functools.partial(jax.jit, static_argnums=(2, 3))
def _embed_gather(idx, table, n_i, n_j):
    ch = _IPC * n_j
    row_w = n_j * _D
    ipw = n_i // _NW
    bpw = ipw * n_j
    nch = ipw // _IPC
    qn = (n_j * _D) // _L
    mesh = plsc.VectorSubcoreMesh(core_axis_name="c", subcore_axis_name="s")

    @functools.partial(
        pl.kernel,
        out_type=jax.ShapeDtypeStruct((n_i, row_w), jnp.float32),
        mesh=mesh,
        scratch_types=[
            pltpu.VMEM((bpw,), jnp.int32),
            pltpu.VMEM((_NBUF, ch, _D), jnp.float32),
            pltpu.VMEM((_NBUF, _IPC, row_w), jnp.float32),
        ] + [pltpu.SemaphoreType.DMA] * _NBUF,
        compiler_params=pltpu.CompilerParams(use_tc_tiling_on_sc=False),
    )
    def k(idx_hbm, table_hbm, out_hbm, idx_v, gbuf, wbuf, *sems):
        wid = lax.axis_index("s") * _NC + lax.axis_index("c")
        base = wid * bpw
        i_base = wid * ipw
        pltpu.sync_copy(idx_hbm.at[pl.ds(base, bpw)], idx_v)

        def start_gather(buf, c):
            pltpu.async_copy(table_hbm.at[idx_v.at[pl.ds(c * ch, ch)]],
                             gbuf.at[buf], sems[buf])

        def wait_gather(buf):
            pltpu.make_async_copy(table_hbm.at[idx_v.at[pl.ds(0, ch)]],
                                  gbuf.at[buf], sems[buf]).wait()

        def shuffle(buf):
            @pl.loop(0, qn, unroll=4)
            def _(q):
                w0 = q * _L
                r_in_i = lax.shift_right_logical(w0, 5)
                c0 = lax.bitwise_and(w0, 31)
                for i in range(_IPC):
                    v = gbuf[buf, i * n_j + r_in_i, pl.ds(c0, _L)]
                    wbuf[buf, i, pl.ds(w0, _L)] = v

        for buf in range(_NBUF):
            start_gather(buf, buf)

        @pl.loop(0, nch, step=_NBUF)
        def _(t):
            for buf in range(_NBUF):
                c = t + buf
                wait_gather(buf)
                shuffle(buf)
                pltpu.sync_copy(wbuf.at[buf],
                                out_hbm.at[pl.ds(i_base + c * _IPC, _IPC)])

                @pl.when(c + _NBUF < nch)
                def _():
                    start_gather(buf, c + _NBUF)

    return k(idx, table)


def kernel(x, table):
    n_i, n_j = x.shape
    out = _embed_gather(x.reshape(-1), table, n_i, n_j)
    return out.reshape(n_i, n_j, table.shape[1])
